# baseline (speedup 1.0000x reference)
import functools

import jax
import jax.numpy as jnp
from jax import lax
from jax.experimental import pallas as pl
from jax.experimental.pallas import tpu as pltpu
from jax.experimental.pallas import tpu_sc as plsc

_EMB_DIM = 64
_GATHER = 128
_K = 8
_CHUNK = _GATHER * _K


@functools.lru_cache(maxsize=None)
def _build(total: int):
    info = plsc.get_sparse_core_info()
    nw = info.num_cores * info.num_subcores
    per_w = total // nw
    n_steps = per_w // _CHUNK
    assert per_w % _CHUNK == 0

    mesh = plsc.VectorSubcoreMesh(core_axis_name="c", subcore_axis_name="s")

    @functools.partial(
        pl.kernel,
        mesh=mesh,
        out_type=jax.ShapeDtypeStruct((total, _EMB_DIM), jnp.float32),
        scratch_types=[
            pltpu.VMEM((_CHUNK,), jnp.int32),
            pltpu.VMEM((_CHUNK, _EMB_DIM), jnp.float32),
            pltpu.SemaphoreType.DMA,
        ],
        compiler_params=pltpu.CompilerParams(use_tc_tiling_on_sc=False),
    )
    def gather_kernel(idx_hbm, table_hbm, out_hbm, idx_v, rows_v, sem):
        wid = lax.axis_index("s") * info.num_cores + lax.axis_index("c")
        base = wid * per_w

        def step(i, carry):
            off = base + i * _CHUNK
            pltpu.sync_copy(idx_hbm.at[pl.ds(off, _CHUNK)], idx_v)
            copies = []
            for j in range(_K):
                copies.append(
                    pltpu.async_copy(
                        table_hbm.at[idx_v.at[pl.ds(j * _GATHER, _GATHER)]],
                        rows_v.at[pl.ds(j * _GATHER, _GATHER)],
                        sem,
                    )
                )
            for c in copies:
                c.wait()
            pltpu.sync_copy(rows_v, out_hbm.at[pl.ds(off, _CHUNK)])
            return carry

        lax.fori_loop(0, n_steps, step, 0)

    return gather_kernel


def kernel(x, m, W):
    b, f = x.shape
    total = b * f
    idx = x.reshape(total).astype(jnp.int32)
    out = _build(total)(idx, W)
    return out.reshape(b, f, _EMB_DIM)


# speedup vs baseline: 1.0101x; 1.0101x over previous
import functools

import jax
import jax.numpy as jnp
from jax import lax
from jax.experimental import pallas as pl
from jax.experimental.pallas import tpu as pltpu
from jax.experimental.pallas import tpu_sc as plsc

_EMB_DIM = 64
_G = 128
_NSLOT = 8


@functools.lru_cache(maxsize=None)
def _build(total: int):
    info = plsc.get_sparse_core_info()
    nw = info.num_cores * info.num_subcores
    per_w = total // nw
    n_chunks = per_w // _G
    n_rounds = n_chunks // _NSLOT
    assert per_w % _G == 0 and n_chunks % _NSLOT == 0 and n_rounds >= 2

    mesh = plsc.VectorSubcoreMesh(core_axis_name="c", subcore_axis_name="s")

    @functools.partial(
        pl.kernel,
        mesh=mesh,
        out_type=jax.ShapeDtypeStruct((total, _EMB_DIM), jnp.float32),
        scratch_types=(
            [pltpu.VMEM((per_w,), jnp.int32)]
            + [pltpu.VMEM((_G, _EMB_DIM), jnp.float32) for _ in range(_NSLOT)]
            + [pltpu.SemaphoreType.DMA for _ in range(_NSLOT)]
        ),
        compiler_params=pltpu.CompilerParams(use_tc_tiling_on_sc=False),
    )
    def gather_kernel(idx_hbm, table_hbm, out_hbm, idx_v, *rest):
        slots = rest[:_NSLOT]
        gsems = rest[_NSLOT:]
        wid = lax.axis_index("s") * info.num_cores + lax.axis_index("c")
        base = wid * per_w

        pltpu.sync_copy(idx_hbm.at[pl.ds(base, per_w)], idx_v)

        def fire(c, s):
            pltpu.async_copy(
                table_hbm.at[idx_v.at[pl.ds(c * _G, _G)]], slots[s], gsems[s]
            )

        def drain(s):
            pltpu.make_async_copy(
                table_hbm.at[idx_v.at[pl.ds(0, _G)]], slots[s], gsems[s]
            ).wait()

        for s in range(_NSLOT):
            fire(s, s)

        def round_body(r, carry):
            for s in range(_NSLOT):
                c = r * _NSLOT + s
                drain(s)
                pltpu.sync_copy(slots[s], out_hbm.at[pl.ds(base + c * _G, _G)])
                fire(c + _NSLOT, s)
            return carry

        lax.fori_loop(0, n_rounds - 1, round_body, 0)

        for s in range(_NSLOT):
            c = (n_rounds - 1) * _NSLOT + s
            drain(s)
            pltpu.sync_copy(slots[s], out_hbm.at[pl.ds(base + c * _G, _G)])

    return gather_kernel


def kernel(x, m, W):
    b, f = x.shape
    total = b * f
    idx = x.reshape(total).astype(jnp.int32)
    out = _build(total)(idx, W)
    return out.reshape(b, f, _EMB_DIM)
